# baseline (speedup 1.0000x reference)
import jax
import jax.numpy as jnp
from jax.experimental import pallas as pl
from jax.experimental.pallas import tpu as pltpu

N = 8192
D = 128
DA = 2 * D

BR = 128


def _gat_kernel(adj_l_ref, adj_r_ref, h_ref, w_ref, wl_ref, wrt_ref, b_ref,
                out_ref, xa_ref, ea_ref, ec_ref, eb_ref, ed_ref):
    r = pl.program_id(0)

    @pl.when(r == 0)
    def _proj():
        x = jnp.dot(h_ref[...], w_ref[...],
                    preferred_element_type=jnp.float32)
        xa_ref[:, :D] = x.astype(jnp.bfloat16)
        col = jax.lax.broadcasted_iota(jnp.int32, (N, D), 1)
        xa_ref[:, D:] = jnp.where(col == 0, 1.0, 0.0).astype(jnp.bfloat16)
        el = jnp.dot(x, wl_ref[...], preferred_element_type=jnp.float32)
        ea_ref[...] = jnp.exp(el).astype(jnp.bfloat16)
        ec_ref[...] = jnp.exp(0.2 * el).astype(jnp.bfloat16)
        er_t = jax.lax.dot_general(wrt_ref[...], x,
                                   dimension_numbers=(((1,), (1,)), ((), ())),
                                   preferred_element_type=jnp.float32)
        eb_ref[...] = jnp.exp(er_t).astype(jnp.bfloat16)
        ed_ref[...] = jnp.exp(0.2 * er_t).astype(jnp.bfloat16)

    @pl.when(r > 0)
    def _rows():
        rb = r - 1
        ea = ea_ref[pl.ds(rb * BR, BR), :]
        ec = ec_ref[pl.ds(rb * BR, BR), :]
        eb = eb_ref[...]
        ed = ed_ref[...]

        HB = N // 2
        zero = jnp.zeros((), jnp.bfloat16)
        p_l = jnp.maximum(ea * eb[:, :HB], ec * ed[:, :HB])
        a_l = jnp.where(adj_l_ref[...] > 0, p_l, zero)
        acc = jnp.dot(a_l, xa_ref[:HB, :],
                      preferred_element_type=jnp.float32)
        p_r = jnp.maximum(ea * eb[:, HB:], ec * ed[:, HB:])
        a_r = jnp.where(adj_r_ref[...] > 0, p_r, zero)
        acc += jnp.dot(a_r, xa_ref[HB:, :],
                       preferred_element_type=jnp.float32)

        num = acc[:, :D]
        den = acc[:, D:D + 1]
        out_ref[...] = num / jnp.maximum(den, 1e-12) + b_ref[...]


@jax.jit
def kernel(h, adj, W, Wl, Wr, b):
    n, d = h.shape

    def rowblk(r):
        return jnp.maximum(r - 1, 0)

    out = pl.pallas_call(
        _gat_kernel,
        grid=(n // BR + 1,),
        in_specs=[
            pl.BlockSpec((BR, n // 2), lambda r: (rowblk(r), 0)),
            pl.BlockSpec((BR, n // 2), lambda r: (rowblk(r), 1)),
            pl.BlockSpec((n, d), lambda r: (0, 0)),
            pl.BlockSpec((d, d), lambda r: (0, 0)),
            pl.BlockSpec((d, 1), lambda r: (0, 0)),
            pl.BlockSpec((1, d), lambda r: (0, 0)),
            pl.BlockSpec((1, d), lambda r: (0, 0)),
        ],
        out_specs=pl.BlockSpec((BR, d), lambda r: (rowblk(r), 0)),
        out_shape=jax.ShapeDtypeStruct((n, d), jnp.float32),
        scratch_shapes=[
            pltpu.VMEM((n, DA), jnp.bfloat16),
            pltpu.VMEM((n, 1), jnp.bfloat16),
            pltpu.VMEM((n, 1), jnp.bfloat16),
            pltpu.VMEM((1, n), jnp.bfloat16),
            pltpu.VMEM((1, n), jnp.bfloat16),
        ],
        compiler_params=pltpu.CompilerParams(
            dimension_semantics=("arbitrary",),
        ),
    )(adj, adj, h, W, Wl, Wr.T, b.reshape(1, d))
    return out


# speedup vs baseline: 1.2600x; 1.2600x over previous
import jax
import jax.numpy as jnp
from jax.experimental import pallas as pl
from jax.experimental.pallas import tpu as pltpu

N = 8192
D = 128
DA = 2 * D

BR = 256


def _gat_kernel(adj_l_ref, adj_r_ref, h_ref, w_ref, wl_ref, wrt_ref, b_ref,
                out_ref, xa_ref, ea_ref, ec_ref, eb_ref, ed_ref):
    r = pl.program_id(0)

    @pl.when(r == 0)
    def _proj():
        x = jnp.dot(h_ref[...], w_ref[...],
                    preferred_element_type=jnp.float32)
        xa_ref[:, :D] = x.astype(jnp.bfloat16)
        col = jax.lax.broadcasted_iota(jnp.int32, (N, D), 1)
        xa_ref[:, D:] = jnp.where(col == 0, 1.0, 0.0).astype(jnp.bfloat16)
        el = jnp.dot(x, wl_ref[...], preferred_element_type=jnp.float32)
        ea_ref[...] = jnp.exp(el).astype(jnp.bfloat16)
        ec_ref[...] = jnp.exp(0.2 * el).astype(jnp.bfloat16)
        er_t = jax.lax.dot_general(wrt_ref[...], x,
                                   dimension_numbers=(((1,), (1,)), ((), ())),
                                   preferred_element_type=jnp.float32)
        eb_ref[...] = jnp.exp(er_t).astype(jnp.bfloat16)
        ed_ref[...] = jnp.exp(0.2 * er_t).astype(jnp.bfloat16)

    @pl.when(r > 0)
    def _rows():
        rb = r - 1
        ea = ea_ref[pl.ds(rb * BR, BR), :]
        ec = ec_ref[pl.ds(rb * BR, BR), :]
        eb = eb_ref[...]
        ed = ed_ref[...]

        HB = N // 2
        zero = jnp.zeros((), jnp.bfloat16)
        p_l = jnp.maximum(ea * eb[:, :HB], ec * ed[:, :HB])
        a_l = jnp.where(adj_l_ref[...] > 0, p_l, zero)
        acc = jnp.dot(a_l, xa_ref[:HB, :],
                      preferred_element_type=jnp.float32)
        p_r = jnp.maximum(ea * eb[:, HB:], ec * ed[:, HB:])
        a_r = jnp.where(adj_r_ref[...] > 0, p_r, zero)
        acc += jnp.dot(a_r, xa_ref[HB:, :],
                       preferred_element_type=jnp.float32)

        num = acc[:, :D]
        den = acc[:, D:D + 1]
        out_ref[...] = num / jnp.maximum(den, 1e-12) + b_ref[...]


@jax.jit
def kernel(h, adj, W, Wl, Wr, b):
    n, d = h.shape

    def rowblk(r):
        return jnp.maximum(r - 1, 0)

    out = pl.pallas_call(
        _gat_kernel,
        grid=(n // BR + 1,),
        in_specs=[
            pl.BlockSpec((BR, n // 2), lambda r: (rowblk(r), 0)),
            pl.BlockSpec((BR, n // 2), lambda r: (rowblk(r), 1)),
            pl.BlockSpec((n, d), lambda r: (0, 0)),
            pl.BlockSpec((d, d), lambda r: (0, 0)),
            pl.BlockSpec((d, 1), lambda r: (0, 0)),
            pl.BlockSpec((1, d), lambda r: (0, 0)),
            pl.BlockSpec((1, d), lambda r: (0, 0)),
        ],
        out_specs=pl.BlockSpec((BR, d), lambda r: (rowblk(r), 0)),
        out_shape=jax.ShapeDtypeStruct((n, d), jnp.float32),
        scratch_shapes=[
            pltpu.VMEM((n, DA), jnp.bfloat16),
            pltpu.VMEM((n, 1), jnp.bfloat16),
            pltpu.VMEM((n, 1), jnp.bfloat16),
            pltpu.VMEM((1, n), jnp.bfloat16),
            pltpu.VMEM((1, n), jnp.bfloat16),
        ],
        compiler_params=pltpu.CompilerParams(
            dimension_semantics=("arbitrary",),
        ),
    )(adj, adj, h, W, Wl, Wr.T, b.reshape(1, d))
    return out
